# baseline (speedup 1.0000x reference)
import functools

import jax
import jax.numpy as jnp
from jax import lax
from jax.experimental import pallas as pl
from jax.experimental.pallas import tpu as pltpu
from jax.experimental.pallas import tpu_sc as plsc

N = 16384
D = 256
LAYERS = 3
NC = 2
NS = 16
DC = 8
DCW = D // DC
BLK = 128
SUP = 32
NBUF = 4
ROWS_PER_TILE = N // NS
SUB = 16
SUB_ROWS = ROWS_PER_TILE // SUB


def _sc_body(nblk_per_tile, edges, vals, xin, fin,
             acc, xtab, est, est_v, g0, g1, g2, g3, zbuf, fbuf,
             gs0, gs1, gs2, gs3, ss0, ss1, ss2, ss3):
    cid = lax.axis_index("c")
    sid = lax.axis_index("s")
    gbufs = (g0, g1, g2, g3)
    gsems = (gs0, gs1, gs2, gs3)
    ssems = (ss0, ss1, ss2, ss3)
    nquad = nblk_per_tile // NBUF
    quads_per_sup = SUP // NBUF

    def zero_zbuf():
        zv = jnp.zeros((16,), jnp.float32)

        @plsc.parallel_loop(0, SUB_ROWS, 1, unroll=2)
        def zrow(r):
            for k in range(DCW // 16):
                zbuf[r, pl.ds(k * 16, 16)] = zv

    def add_zbuf_into_fbuf():
        @plsc.parallel_loop(0, SUB_ROWS, 1, unroll=2)
        def arow(r):
            for k in range(DCW // 16):
                sl = pl.ds(k * 16, 16)
                fbuf[r, sl] = fbuf[r, sl] + zbuf[r, sl]

    def chunk_body(cc, _):
        c = cid * (DC // NC) + cc

        def init_ss(ss, _):
            r0 = sid * ROWS_PER_TILE + ss * SUB_ROWS
            sl = pl.ds(r0, SUB_ROWS)
            pltpu.sync_copy(xin.at[c].at[sl], fbuf)
            pltpu.sync_copy(fbuf, fin.at[c].at[sl])
            pltpu.sync_copy(fbuf, xtab.at[sl])
            return 0
        lax.fori_loop(0, SUB, init_ss, 0)
        plsc.subcore_barrier()

        def layer_body(_layer, __):
            zero_zbuf()

            def zero_ss(ss, _):
                r0 = sid * ROWS_PER_TILE + ss * SUB_ROWS
                pltpu.sync_copy(zbuf, acc.at[pl.ds(r0, SUB_ROWS)])
                return 0
            lax.fori_loop(0, SUB, zero_ss, 0)
            plsc.subcore_barrier()

            blk_base = sid * nblk_per_tile

            def stage(sup_idx, buf_idx):
                pltpu.sync_copy(edges.at[pl.ds(blk_base + sup_idx * SUP, SUP)],
                                est.at[buf_idx])
                pltpu.sync_copy(vals.at[pl.ds(blk_base + sup_idx * SUP, SUP)],
                                est_v.at[buf_idx])

            def start_gather(par, j, t):
                pltpu.async_copy(xtab.at[est.at[par, j, 0]],
                                 gbufs[t], gsems[t])

            def scale(par, j, t):
                g = gbufs[t]

                @plsc.parallel_loop(0, BLK // 16, 1, unroll=2)
                def m_body(m):
                    vv = est_v[par, j, pl.ds(m * 16, 16)]
                    base = m * 16
                    for tt in range(16):
                        v = vv[tt]
                        i = base + tt
                        for k in range(DCW // 16):
                            sl = pl.ds(k * 16, 16)
                            g[i, sl] = g[i, sl] * v

            stage(0, 0)
            start_gather(0, 0, 0)
            start_gather(0, 1, 1)

            def quad(p, _):
                for t in range(NBUF):
                    k = p * NBUF + t
                    j = jnp.bitwise_and(k, SUP - 1)
                    par = jnp.bitwise_and(jnp.right_shift(k, 5), 1)
                    t2 = (t + 2) % NBUF
                    pltpu.make_async_copy(
                        xtab.at[est.at[par, j, 0]], gbufs[t],
                        gsems[t]).wait()
                    scale(par, j, t)
                    pltpu.async_copy(gbufs[t], acc.at[est.at[par, j, 1]],
                                     ssems[t], add=True)
                    if t < 2:
                        @pl.when(p > 0)
                        def _():
                            pltpu.make_async_copy(
                                gbufs[t2], acc.at[est.at[par, j, 1]],
                                ssems[t2]).wait()
                    else:
                        pltpu.make_async_copy(
                            gbufs[t2], acc.at[est.at[par, j, 1]],
                            ssems[t2]).wait()
                    if t == 2:
                        @pl.when(jnp.logical_and(
                            jnp.bitwise_and(p, quads_per_sup - 1)
                            == quads_per_sup - 1,
                            p < nquad - 1))
                        def _():
                            stage(jnp.right_shift(p, 3) + 1,
                                  jnp.bitwise_and(jnp.right_shift(p, 3) + 1,
                                                  1))
                    kn = k + 2
                    jn = jnp.bitwise_and(kn, SUP - 1)
                    parn = jnp.bitwise_and(jnp.right_shift(kn, 5), 1)
                    if t < 2:
                        start_gather(parn, jn, t2)
                    else:
                        @pl.when(p < nquad - 1)
                        def _():
                            start_gather(parn, jn, t2)
                return 0
            lax.fori_loop(0, nquad, quad, 0)
            pltpu.make_async_copy(gbufs[2], acc.at[est.at[0, 0, 1]],
                                  ssems[2]).wait()
            pltpu.make_async_copy(gbufs[3], acc.at[est.at[0, 0, 1]],
                                  ssems[3]).wait()
            plsc.subcore_barrier()

            def end_ss(ss, _):
                r0 = sid * ROWS_PER_TILE + ss * SUB_ROWS
                sl = pl.ds(r0, SUB_ROWS)
                pltpu.sync_copy(acc.at[pl.ds(r0, SUB_ROWS)], zbuf)
                pltpu.sync_copy(fin.at[c].at[sl], fbuf)
                add_zbuf_into_fbuf()
                pltpu.sync_copy(fbuf, fin.at[c].at[sl])
                pltpu.sync_copy(zbuf, xtab.at[sl])
                return 0
            lax.fori_loop(0, SUB, end_ss, 0)
            plsc.subcore_barrier()
            return 0
        lax.fori_loop(0, LAYERS, layer_body, 0)
        return 0
    lax.fori_loop(0, DC // NC, chunk_body, 0)


def kernel(values, row, col, embedding):
    values = values.astype(jnp.float32)
    row = row.astype(jnp.int32)
    col = col.astype(jnp.int32)
    embedding = embedding.astype(jnp.float32)

    nnz = values.shape[0]
    per_tile_edges = -(-nnz // (NS * SUP * BLK)) * SUP * BLK
    tot = per_tile_edges * NS
    pad = tot - nnz
    values = jnp.pad(values, (0, pad))
    row = jnp.pad(row, (0, pad))
    col = jnp.pad(col, (0, pad))
    edges = jnp.stack(
        [col.reshape(tot // BLK, BLK),
         row.reshape(tot // BLK, BLK)],
        axis=1)
    vals2d = values.reshape(tot // BLK, BLK)

    xin = jnp.transpose(embedding.reshape(N, DC, DCW), (1, 0, 2))

    nblk_per_tile = per_tile_edges // BLK

    grid_kernel = functools.partial(
        pl.kernel,
        mesh=plsc.VectorSubcoreMesh(core_axis_name="c", subcore_axis_name="s"),
        compiler_params=pltpu.CompilerParams(use_tc_tiling_on_sc=False),
        out_type=jax.ShapeDtypeStruct((DC, N, DCW), jnp.float32),
        scratch_types=[
            pltpu.VMEM_SHARED((N, DCW), jnp.float32),
            pltpu.VMEM_SHARED((N, DCW), jnp.float32),
            pltpu.VMEM((2, SUP, 2, BLK), jnp.int32),
            pltpu.VMEM((2, SUP, BLK), jnp.float32),
            pltpu.VMEM((BLK, DCW), jnp.float32),
            pltpu.VMEM((BLK, DCW), jnp.float32),
            pltpu.VMEM((BLK, DCW), jnp.float32),
            pltpu.VMEM((BLK, DCW), jnp.float32),
            pltpu.VMEM((SUB_ROWS, DCW), jnp.float32),
            pltpu.VMEM((SUB_ROWS, DCW), jnp.float32),
            pltpu.SemaphoreType.DMA,
            pltpu.SemaphoreType.DMA,
            pltpu.SemaphoreType.DMA,
            pltpu.SemaphoreType.DMA,
            pltpu.SemaphoreType.DMA,
            pltpu.SemaphoreType.DMA,
            pltpu.SemaphoreType.DMA,
            pltpu.SemaphoreType.DMA,
        ],
    )(functools.partial(_sc_body, nblk_per_tile))

    fin = grid_kernel(edges, vals2d, xin)
    return jnp.transpose(fin, (1, 0, 2)).reshape(N, D)


# speedup vs baseline: 1.0556x; 1.0556x over previous
import functools

import jax
import jax.numpy as jnp
from jax import lax
from jax.experimental import pallas as pl
from jax.experimental.pallas import tpu as pltpu
from jax.experimental.pallas import tpu_sc as plsc

N = 16384
D = 256
LAYERS = 3
NC = 2
NS = 16
DC = 8
DCW = D // DC
BLK = 128
SUP = 32
NBUF = 4
ROWS_PER_TILE = N // NS
SUB = 16
SUB_ROWS = ROWS_PER_TILE // SUB


def _sc_body(nblk_per_tile, edges, vals, xin, fin,
             acc, xtab, est, est_v, g0, g1, g2, g3, zbuf, fbuf,
             gs0, gs1, gs2, gs3, ss0, ss1, ss2, ss3, stsem):
    cid = lax.axis_index("c")
    sid = lax.axis_index("s")
    gbufs = (g0, g1, g2, g3)
    gsems = (gs0, gs1, gs2, gs3)
    ssems = (ss0, ss1, ss2, ss3)
    nquad = nblk_per_tile // NBUF
    nsup = nblk_per_tile // SUP
    quads_per_sup = SUP // NBUF

    def zero_zbuf():
        zv = jnp.zeros((16,), jnp.float32)

        @plsc.parallel_loop(0, SUB_ROWS, 1, unroll=2)
        def zrow(r):
            for k in range(DCW // 16):
                zbuf[r, pl.ds(k * 16, 16)] = zv

    def add_zbuf_into_fbuf():
        @plsc.parallel_loop(0, SUB_ROWS, 1, unroll=2)
        def arow(r):
            for k in range(DCW // 16):
                sl = pl.ds(k * 16, 16)
                fbuf[r, sl] = fbuf[r, sl] + zbuf[r, sl]

    def chunk_body(cc, _):
        c = cid * (DC // NC) + cc

        def init_ss(ss, _):
            r0 = sid * ROWS_PER_TILE + ss * SUB_ROWS
            sl = pl.ds(r0, SUB_ROWS)
            pltpu.sync_copy(xin.at[c].at[sl], fbuf)
            pltpu.sync_copy(fbuf, fin.at[c].at[sl])
            pltpu.sync_copy(fbuf, xtab.at[sl])
            return 0
        lax.fori_loop(0, SUB, init_ss, 0)
        plsc.subcore_barrier()

        def layer_body(_layer, __):
            zero_zbuf()

            def zero_ss(ss, _):
                r0 = sid * ROWS_PER_TILE + ss * SUB_ROWS
                pltpu.sync_copy(zbuf, acc.at[pl.ds(r0, SUB_ROWS)])
                return 0
            lax.fori_loop(0, SUB, zero_ss, 0)
            plsc.subcore_barrier()

            blk_base = sid * nblk_per_tile

            def stage(sup_idx, buf_idx):
                pltpu.sync_copy(edges.at[pl.ds(blk_base + sup_idx * SUP, SUP)],
                                est.at[buf_idx])
                pltpu.sync_copy(vals.at[pl.ds(blk_base + sup_idx * SUP, SUP)],
                                est_v.at[buf_idx])

            def stage_async(sup_idx, buf_idx):
                pltpu.async_copy(
                    edges.at[pl.ds(blk_base + sup_idx * SUP, SUP)],
                    est.at[buf_idx], stsem)
                pltpu.async_copy(
                    vals.at[pl.ds(blk_base + sup_idx * SUP, SUP)],
                    est_v.at[buf_idx], stsem)

            def stage_wait(sup_idx, buf_idx):
                pltpu.make_async_copy(
                    edges.at[pl.ds(blk_base + sup_idx * SUP, SUP)],
                    est.at[buf_idx], stsem).wait()
                pltpu.make_async_copy(
                    vals.at[pl.ds(blk_base + sup_idx * SUP, SUP)],
                    est_v.at[buf_idx], stsem).wait()

            def start_gather(par, j, t):
                pltpu.async_copy(xtab.at[est.at[par, j, 0]],
                                 gbufs[t], gsems[t])

            def scale(par, j, t):
                g = gbufs[t]

                @plsc.parallel_loop(0, BLK // 16, 1, unroll=4)
                def m_body(m):
                    vv = est_v[par, j, pl.ds(m * 16, 16)]
                    base = m * 16
                    for tt in range(16):
                        v = vv[tt]
                        i = base + tt
                        for k in range(DCW // 16):
                            sl = pl.ds(k * 16, 16)
                            g[i, sl] = g[i, sl] * v

            stage(0, 0)
            start_gather(0, 0, 0)
            start_gather(0, 1, 1)

            def quad(p, _):
                for t in range(NBUF):
                    k = p * NBUF + t
                    j = jnp.bitwise_and(k, SUP - 1)
                    par = jnp.bitwise_and(jnp.right_shift(k, 5), 1)
                    t2 = (t + 2) % NBUF
                    pltpu.make_async_copy(
                        xtab.at[est.at[par, j, 0]], gbufs[t],
                        gsems[t]).wait()
                    scale(par, j, t)
                    pltpu.async_copy(gbufs[t], acc.at[est.at[par, j, 1]],
                                     ssems[t], add=True)
                    if t < 2:
                        @pl.when(p > 0)
                        def _():
                            pltpu.make_async_copy(
                                gbufs[t2], acc.at[est.at[par, j, 1]],
                                ssems[t2]).wait()
                    else:
                        pltpu.make_async_copy(
                            gbufs[t2], acc.at[est.at[par, j, 1]],
                            ssems[t2]).wait()
                    if t == 2:
                        @pl.when(jnp.logical_and(
                            jnp.bitwise_and(p, quads_per_sup - 1) == 2,
                            jnp.right_shift(p, 3) < nsup - 1))
                        def _():
                            sup1 = jnp.right_shift(p, 3) + 1
                            stage_async(sup1, jnp.bitwise_and(sup1, 1))

                        @pl.when(jnp.logical_and(
                            jnp.bitwise_and(p, quads_per_sup - 1)
                            == quads_per_sup - 1,
                            p < nquad - 1))
                        def _():
                            sup1 = jnp.right_shift(p, 3) + 1
                            stage_wait(sup1, jnp.bitwise_and(sup1, 1))
                    kn = k + 2
                    jn = jnp.bitwise_and(kn, SUP - 1)
                    parn = jnp.bitwise_and(jnp.right_shift(kn, 5), 1)
                    if t < 2:
                        start_gather(parn, jn, t2)
                    else:
                        @pl.when(p < nquad - 1)
                        def _():
                            start_gather(parn, jn, t2)
                return 0
            lax.fori_loop(0, nquad, quad, 0)
            pltpu.make_async_copy(gbufs[2], acc.at[est.at[0, 0, 1]],
                                  ssems[2]).wait()
            pltpu.make_async_copy(gbufs[3], acc.at[est.at[0, 0, 1]],
                                  ssems[3]).wait()
            plsc.subcore_barrier()

            def end_ss(ss, _):
                r0 = sid * ROWS_PER_TILE + ss * SUB_ROWS
                sl = pl.ds(r0, SUB_ROWS)
                pltpu.sync_copy(acc.at[pl.ds(r0, SUB_ROWS)], zbuf)
                pltpu.sync_copy(fin.at[c].at[sl], fbuf)
                add_zbuf_into_fbuf()
                pltpu.sync_copy(fbuf, fin.at[c].at[sl])
                pltpu.sync_copy(zbuf, xtab.at[sl])
                return 0
            lax.fori_loop(0, SUB, end_ss, 0)
            plsc.subcore_barrier()
            return 0
        lax.fori_loop(0, LAYERS, layer_body, 0)
        return 0
    lax.fori_loop(0, DC // NC, chunk_body, 0)


def kernel(values, row, col, embedding):
    values = values.astype(jnp.float32)
    row = row.astype(jnp.int32)
    col = col.astype(jnp.int32)
    embedding = embedding.astype(jnp.float32)

    nnz = values.shape[0]
    per_tile_edges = -(-nnz // (NS * SUP * BLK)) * SUP * BLK
    tot = per_tile_edges * NS
    pad = tot - nnz
    values = jnp.pad(values, (0, pad))
    row = jnp.pad(row, (0, pad))
    col = jnp.pad(col, (0, pad))
    edges = jnp.stack(
        [col.reshape(tot // BLK, BLK),
         row.reshape(tot // BLK, BLK)],
        axis=1)
    vals2d = values.reshape(tot // BLK, BLK)

    xin = jnp.transpose(embedding.reshape(N, DC, DCW), (1, 0, 2))

    nblk_per_tile = per_tile_edges // BLK

    grid_kernel = functools.partial(
        pl.kernel,
        mesh=plsc.VectorSubcoreMesh(core_axis_name="c", subcore_axis_name="s"),
        compiler_params=pltpu.CompilerParams(use_tc_tiling_on_sc=False),
        out_type=jax.ShapeDtypeStruct((DC, N, DCW), jnp.float32),
        scratch_types=[
            pltpu.VMEM_SHARED((N, DCW), jnp.float32),
            pltpu.VMEM_SHARED((N, DCW), jnp.float32),
            pltpu.VMEM((2, SUP, 2, BLK), jnp.int32),
            pltpu.VMEM((2, SUP, BLK), jnp.float32),
            pltpu.VMEM((BLK, DCW), jnp.float32),
            pltpu.VMEM((BLK, DCW), jnp.float32),
            pltpu.VMEM((BLK, DCW), jnp.float32),
            pltpu.VMEM((BLK, DCW), jnp.float32),
            pltpu.VMEM((SUB_ROWS, DCW), jnp.float32),
            pltpu.VMEM((SUB_ROWS, DCW), jnp.float32),
            pltpu.SemaphoreType.DMA,
            pltpu.SemaphoreType.DMA,
            pltpu.SemaphoreType.DMA,
            pltpu.SemaphoreType.DMA,
            pltpu.SemaphoreType.DMA,
            pltpu.SemaphoreType.DMA,
            pltpu.SemaphoreType.DMA,
            pltpu.SemaphoreType.DMA,
            pltpu.SemaphoreType.DMA,
        ],
    )(functools.partial(_sc_body, nblk_per_tile))

    fin = grid_kernel(edges, vals2d, xin)
    return jnp.transpose(fin, (1, 0, 2)).reshape(N, D)


# speedup vs baseline: 1.1226x; 1.0636x over previous
import functools

import jax
import jax.numpy as jnp
from jax import lax
from jax.experimental import pallas as pl
from jax.experimental.pallas import tpu as pltpu
from jax.experimental.pallas import tpu_sc as plsc

N = 16384
D = 256
LAYERS = 3
NC = 2
NS = 16
DC = 8
DCW = D // DC
BLK = 128
SUP = 32
NBUF = 8
ROWS_PER_TILE = N // NS
SUB = 16
SUB_ROWS = ROWS_PER_TILE // SUB


def _sc_body(nblk_per_tile, edges, vals, xin, fin,
             acc, xtab, est, est_v,
             g0, g1, g2, g3, g4, g5, g6, g7, zbuf, fbuf,
             gs0, gs1, gs2, gs3, gs4, gs5, gs6, gs7,
             ss0, ss1, ss2, ss3, ss4, ss5, ss6, ss7, stsem):
    cid = lax.axis_index("c")
    sid = lax.axis_index("s")
    gbufs = (g0, g1, g2, g3, g4, g5, g6, g7)
    gsems = (gs0, gs1, gs2, gs3, gs4, gs5, gs6, gs7)
    ssems = (ss0, ss1, ss2, ss3, ss4, ss5, ss6, ss7)
    nquad = nblk_per_tile // NBUF
    nsup = nblk_per_tile // SUP
    quads_per_sup = SUP // NBUF

    def zero_zbuf():
        zv = jnp.zeros((16,), jnp.float32)

        @plsc.parallel_loop(0, SUB_ROWS, 1, unroll=2)
        def zrow(r):
            for k in range(DCW // 16):
                zbuf[r, pl.ds(k * 16, 16)] = zv

    def add_zbuf_into_fbuf():
        @plsc.parallel_loop(0, SUB_ROWS, 1, unroll=2)
        def arow(r):
            for k in range(DCW // 16):
                sl = pl.ds(k * 16, 16)
                fbuf[r, sl] = fbuf[r, sl] + zbuf[r, sl]

    def chunk_body(cc, _):
        c = cid * (DC // NC) + cc

        def init_ss(ss, _):
            r0 = sid * ROWS_PER_TILE + ss * SUB_ROWS
            sl = pl.ds(r0, SUB_ROWS)
            pltpu.sync_copy(xin.at[c].at[sl], fbuf)
            pltpu.sync_copy(fbuf, fin.at[c].at[sl])
            pltpu.sync_copy(fbuf, xtab.at[sl])
            return 0
        lax.fori_loop(0, SUB, init_ss, 0)
        plsc.subcore_barrier()

        def layer_body(_layer, __):
            zero_zbuf()

            def zero_ss(ss, _):
                r0 = sid * ROWS_PER_TILE + ss * SUB_ROWS
                pltpu.sync_copy(zbuf, acc.at[pl.ds(r0, SUB_ROWS)])
                return 0
            lax.fori_loop(0, SUB, zero_ss, 0)
            plsc.subcore_barrier()

            blk_base = sid * nblk_per_tile

            def stage(sup_idx, buf_idx):
                pltpu.sync_copy(edges.at[pl.ds(blk_base + sup_idx * SUP, SUP)],
                                est.at[buf_idx])
                pltpu.sync_copy(vals.at[pl.ds(blk_base + sup_idx * SUP, SUP)],
                                est_v.at[buf_idx])

            def stage_async(sup_idx, buf_idx):
                pltpu.async_copy(
                    edges.at[pl.ds(blk_base + sup_idx * SUP, SUP)],
                    est.at[buf_idx], stsem)
                pltpu.async_copy(
                    vals.at[pl.ds(blk_base + sup_idx * SUP, SUP)],
                    est_v.at[buf_idx], stsem)

            def stage_wait(sup_idx, buf_idx):
                pltpu.make_async_copy(
                    edges.at[pl.ds(blk_base + sup_idx * SUP, SUP)],
                    est.at[buf_idx], stsem).wait()
                pltpu.make_async_copy(
                    vals.at[pl.ds(blk_base + sup_idx * SUP, SUP)],
                    est_v.at[buf_idx], stsem).wait()

            def start_gather(par, j, t):
                pltpu.async_copy(xtab.at[est.at[par, j, 0]],
                                 gbufs[t], gsems[t])

            def scale(par, j, t):
                g = gbufs[t]

                @plsc.parallel_loop(0, BLK // 16, 1, unroll=4)
                def m_body(m):
                    vv = est_v[par, j, pl.ds(m * 16, 16)]
                    base = m * 16
                    for tt in range(16):
                        v = vv[tt]
                        i = base + tt
                        for k in range(DCW // 16):
                            sl = pl.ds(k * 16, 16)
                            g[i, sl] = g[i, sl] * v

            stage(0, 0)
            start_gather(0, 0, 0)
            start_gather(0, 1, 1)
            start_gather(0, 2, 2)
            start_gather(0, 3, 3)

            def octet(p, _):
                for t in range(NBUF):
                    k = p * NBUF + t
                    j = jnp.bitwise_and(k, SUP - 1)
                    par = jnp.bitwise_and(jnp.right_shift(k, 5), 1)
                    t4 = (t + 4) % NBUF
                    pltpu.make_async_copy(
                        xtab.at[est.at[par, j, 0]], gbufs[t],
                        gsems[t]).wait()
                    scale(par, j, t)
                    pltpu.async_copy(gbufs[t], acc.at[est.at[par, j, 1]],
                                     ssems[t], add=True)
                    if t < 4:
                        @pl.when(p > 0)
                        def _():
                            pltpu.make_async_copy(
                                gbufs[t4], acc.at[est.at[par, j, 1]],
                                ssems[t4]).wait()
                    else:
                        pltpu.make_async_copy(
                            gbufs[t4], acc.at[est.at[par, j, 1]],
                            ssems[t4]).wait()
                    if t == 0:
                        @pl.when(jnp.logical_and(
                            jnp.bitwise_and(p, quads_per_sup - 1) == 1,
                            jnp.right_shift(p, 2) < nsup - 1))
                        def _():
                            sup1 = jnp.right_shift(p, 2) + 1
                            stage_async(sup1, jnp.bitwise_and(sup1, 1))
                    if t == 3:
                        @pl.when(jnp.logical_and(
                            jnp.bitwise_and(p, quads_per_sup - 1)
                            == quads_per_sup - 1,
                            p < nquad - 1))
                        def _():
                            sup1 = jnp.right_shift(p, 2) + 1
                            stage_wait(sup1, jnp.bitwise_and(sup1, 1))
                    kn = k + 4
                    jn = jnp.bitwise_and(kn, SUP - 1)
                    parn = jnp.bitwise_and(jnp.right_shift(kn, 5), 1)
                    if t < 4:
                        start_gather(parn, jn, t4)
                    else:
                        @pl.when(p < nquad - 1)
                        def _():
                            start_gather(parn, jn, t4)
                return 0
            lax.fori_loop(0, nquad, octet, 0)
            pltpu.make_async_copy(gbufs[4], acc.at[est.at[0, 0, 1]],
                                  ssems[4]).wait()
            pltpu.make_async_copy(gbufs[5], acc.at[est.at[0, 0, 1]],
                                  ssems[5]).wait()
            pltpu.make_async_copy(gbufs[6], acc.at[est.at[0, 0, 1]],
                                  ssems[6]).wait()
            pltpu.make_async_copy(gbufs[7], acc.at[est.at[0, 0, 1]],
                                  ssems[7]).wait()
            plsc.subcore_barrier()

            def end_ss(ss, _):
                r0 = sid * ROWS_PER_TILE + ss * SUB_ROWS
                sl = pl.ds(r0, SUB_ROWS)
                pltpu.sync_copy(acc.at[pl.ds(r0, SUB_ROWS)], zbuf)
                pltpu.sync_copy(fin.at[c].at[sl], fbuf)
                add_zbuf_into_fbuf()
                pltpu.sync_copy(fbuf, fin.at[c].at[sl])
                pltpu.sync_copy(zbuf, xtab.at[sl])
                return 0
            lax.fori_loop(0, SUB, end_ss, 0)
            plsc.subcore_barrier()
            return 0
        lax.fori_loop(0, LAYERS, layer_body, 0)
        return 0
    lax.fori_loop(0, DC // NC, chunk_body, 0)


def kernel(values, row, col, embedding):
    values = values.astype(jnp.float32)
    row = row.astype(jnp.int32)
    col = col.astype(jnp.int32)
    embedding = embedding.astype(jnp.float32)

    nnz = values.shape[0]
    per_tile_edges = -(-nnz // (NS * SUP * BLK)) * SUP * BLK
    tot = per_tile_edges * NS
    pad = tot - nnz
    values = jnp.pad(values, (0, pad))
    row = jnp.pad(row, (0, pad))
    col = jnp.pad(col, (0, pad))
    edges = jnp.stack(
        [col.reshape(tot // BLK, BLK),
         row.reshape(tot // BLK, BLK)],
        axis=1)
    vals2d = values.reshape(tot // BLK, BLK)

    xin = jnp.transpose(embedding.reshape(N, DC, DCW), (1, 0, 2))

    nblk_per_tile = per_tile_edges // BLK

    grid_kernel = functools.partial(
        pl.kernel,
        mesh=plsc.VectorSubcoreMesh(core_axis_name="c", subcore_axis_name="s"),
        compiler_params=pltpu.CompilerParams(use_tc_tiling_on_sc=False),
        out_type=jax.ShapeDtypeStruct((DC, N, DCW), jnp.float32),
        scratch_types=[
            pltpu.VMEM_SHARED((N, DCW), jnp.float32),
            pltpu.VMEM_SHARED((N, DCW), jnp.float32),
            pltpu.VMEM((2, SUP, 2, BLK), jnp.int32),
            pltpu.VMEM((2, SUP, BLK), jnp.float32),
            pltpu.VMEM((BLK, DCW), jnp.float32),
            pltpu.VMEM((BLK, DCW), jnp.float32),
            pltpu.VMEM((BLK, DCW), jnp.float32),
            pltpu.VMEM((BLK, DCW), jnp.float32),
            pltpu.VMEM((BLK, DCW), jnp.float32),
            pltpu.VMEM((BLK, DCW), jnp.float32),
            pltpu.VMEM((BLK, DCW), jnp.float32),
            pltpu.VMEM((BLK, DCW), jnp.float32),
            pltpu.VMEM((SUB_ROWS, DCW), jnp.float32),
            pltpu.VMEM((SUB_ROWS, DCW), jnp.float32),
            pltpu.SemaphoreType.DMA,
            pltpu.SemaphoreType.DMA,
            pltpu.SemaphoreType.DMA,
            pltpu.SemaphoreType.DMA,
            pltpu.SemaphoreType.DMA,
            pltpu.SemaphoreType.DMA,
            pltpu.SemaphoreType.DMA,
            pltpu.SemaphoreType.DMA,
            pltpu.SemaphoreType.DMA,
            pltpu.SemaphoreType.DMA,
            pltpu.SemaphoreType.DMA,
            pltpu.SemaphoreType.DMA,
            pltpu.SemaphoreType.DMA,
            pltpu.SemaphoreType.DMA,
            pltpu.SemaphoreType.DMA,
            pltpu.SemaphoreType.DMA,
            pltpu.SemaphoreType.DMA,
        ],
    )(functools.partial(_sc_body, nblk_per_tile))

    fin = grid_kernel(edges, vals2d, xin)
    return jnp.transpose(fin, (1, 0, 2)).reshape(N, D)
